# baseline (speedup 1.0000x reference)
import jax
import jax.numpy as jnp
from jax.experimental import pallas as pl
from jax.experimental.pallas import tpu as pltpu

TOKEN_BLOCK = 512
NBUF = 4
SPLIT = 2
ROWS_PER_COPY = TOKEN_BLOCK // SPLIT


def _router_mlp_kernel(x_hbm, w1_ref, b1_ref, w2_ref, b2_ref, o_ref,
                       x_bufs, sems):
    tokens = x_hbm.shape[0]
    n_blocks = tokens // TOKEN_BLOCK

    def copy_for(i, slot, part):
        return pltpu.make_async_copy(
            x_hbm.at[pl.ds(i * TOKEN_BLOCK + part * ROWS_PER_COPY,
                           ROWS_PER_COPY), :],
            x_bufs.at[slot, pl.ds(part * ROWS_PER_COPY, ROWS_PER_COPY), :],
            sems.at[slot, part],
        )

    def start_all(i, slot):
        for p in range(SPLIT):
            copy_for(i, slot, p).start()

    def wait_all(i, slot):
        for p in range(SPLIT):
            copy_for(i, slot, p).wait()

    for j in range(NBUF):
        start_all(j, j)

    def body(i, carry):
        slot = jax.lax.rem(i, NBUF)
        wait_all(i, slot)
        h = jnp.dot(x_bufs[slot], w1_ref[...],
                    preferred_element_type=jnp.float32)
        h = jnp.maximum(h + b1_ref[...], 0.0)
        o_ref[pl.ds(i * TOKEN_BLOCK, TOKEN_BLOCK), :] = (
            jnp.dot(h, w2_ref[...], preferred_element_type=jnp.float32)
            + b2_ref[...]
        )
        nxt = i + NBUF

        @pl.when(nxt < n_blocks)
        def _():
            start_all(nxt, slot)

        return carry

    jax.lax.fori_loop(0, n_blocks, body, 0)


@jax.jit
def kernel(x, W1, b1, W2, b2):
    tokens, input_dim = x.shape
    hidden = W1.shape[1]
    num_experts = W2.shape[1]
    b1 = b1.reshape(1, hidden)
    b2 = b2.reshape(1, num_experts)
    return pl.pallas_call(
        _router_mlp_kernel,
        in_specs=[
            pl.BlockSpec(memory_space=pl.ANY),
            pl.BlockSpec(memory_space=pltpu.VMEM),
            pl.BlockSpec(memory_space=pltpu.VMEM),
            pl.BlockSpec(memory_space=pltpu.VMEM),
            pl.BlockSpec(memory_space=pltpu.VMEM),
        ],
        out_specs=pl.BlockSpec(memory_space=pltpu.VMEM),
        out_shape=jax.ShapeDtypeStruct((tokens, num_experts), jnp.float32),
        scratch_shapes=[
            pltpu.VMEM((NBUF, TOKEN_BLOCK, input_dim), jnp.float32),
            pltpu.SemaphoreType.DMA((NBUF, SPLIT)),
        ],
    )(x, W1, b1, W2, b2)


# speedup vs baseline: 1.0489x; 1.0489x over previous
import jax
import jax.numpy as jnp
from jax.experimental import pallas as pl
from jax.experimental.pallas import tpu as pltpu

TOKEN_BLOCK = 512


def _router_mlp_kernel(x_ref, w1_ref, b1_ref, w2_ref, b2_ref, o_ref):
    h = jnp.dot(x_ref[...], w1_ref[...], preferred_element_type=jnp.float32)
    h = jnp.maximum(h + b1_ref[...], 0.0)
    o_ref[...] = (
        jnp.dot(h, w2_ref[...], preferred_element_type=jnp.float32)
        + b2_ref[...]
    )


@jax.jit
def kernel(x, W1, b1, W2, b2):
    tokens, input_dim = x.shape
    hidden = W1.shape[1]
    num_experts = W2.shape[1]
    b1 = b1.reshape(1, hidden)
    b2 = b2.reshape(1, num_experts)
    grid = (tokens // TOKEN_BLOCK,)
    return pl.pallas_call(
        _router_mlp_kernel,
        grid=grid,
        in_specs=[
            pl.BlockSpec((TOKEN_BLOCK, input_dim), lambda i: (i, 0)),
            pl.BlockSpec((input_dim, hidden), lambda i: (0, 0)),
            pl.BlockSpec((1, hidden), lambda i: (0, 0)),
            pl.BlockSpec((hidden, num_experts), lambda i: (0, 0)),
            pl.BlockSpec((1, num_experts), lambda i: (0, 0)),
        ],
        out_specs=pl.BlockSpec((TOKEN_BLOCK, num_experts), lambda i: (i, 0)),
        out_shape=jax.ShapeDtypeStruct((tokens, num_experts), jnp.float32),
        compiler_params=pltpu.CompilerParams(
            dimension_semantics=("arbitrary",),
            skip_device_barrier=True,
            disable_bounds_checks=True,
            disable_semaphore_checks=True,
        ),
    )(x, W1, b1, W2, b2)
